# baseline (speedup 1.0000x reference)
import functools

import jax
import jax.numpy as jnp
from jax import lax
from jax.experimental import pallas as pl
from jax.experimental.pallas import tpu as pltpu
from jax.experimental.pallas import tpu_sc as plsc

_NC = 2
_NS = 16
_NW = _NC * _NS


def _sc_body(rows_per_w, SD, x_hbm, tbl_hbm, out_hbm,
             tblv, buf0, buf1, tsem, isem0, isem1, osem0, osem1):
    wid = lax.axis_index("s") * _NC + lax.axis_index("c")
    base = wid * rows_per_w
    bufs = (buf0, buf1)
    isems = (isem0, isem1)
    osems = (osem0, osem1)

    pltpu.make_async_copy(tbl_hbm, tblv, tsem).start()
    pltpu.make_async_copy(tbl_hbm, tblv, tsem).wait()

    def in_copy(r):
        return pltpu.make_async_copy(
            x_hbm.at[base + r], bufs[r % 2], isems[r % 2]
        )

    def out_copy(r):
        return pltpu.make_async_copy(
            bufs[r % 2], out_hbm.at[base + r], osems[r % 2]
        )

    in_copy(0).start()
    for r in range(rows_per_w):
        in_copy(r).wait()
        b = bufs[r % 2]

        @plsc.parallel_loop(0, SD, 16, unroll=8)
        def _add(i):
            b[pl.ds(i, 16)] = b[pl.ds(i, 16)] + tblv[pl.ds(i, 16)]

        out_copy(r).start()
        if r + 1 < rows_per_w:
            if r >= 1:
                out_copy(r - 1).wait()
            in_copy(r + 1).start()
    for r in range(max(0, rows_per_w - 2), rows_per_w):
        out_copy(r).wait()


def kernel(x, pos_emb_weight):
    B, S, D = x.shape
    SD = S * D
    rows_per_w = B // _NW
    x2 = x.reshape(B, SD)
    tbl = pos_emb_weight[:S].reshape(SD)
    mesh = plsc.VectorSubcoreMesh(core_axis_name="c", subcore_axis_name="s")
    body = functools.partial(_sc_body, rows_per_w, SD)
    out = pl.kernel(
        body,
        out_type=jax.ShapeDtypeStruct((B, SD), jnp.float32),
        mesh=mesh,
        scratch_types=[
            pltpu.VMEM((SD,), jnp.float32),
            pltpu.VMEM((SD,), jnp.float32),
            pltpu.VMEM((SD,), jnp.float32),
            pltpu.SemaphoreType.DMA,
            pltpu.SemaphoreType.DMA,
            pltpu.SemaphoreType.DMA,
            pltpu.SemaphoreType.DMA,
            pltpu.SemaphoreType.DMA,
        ],
    )(x2, tbl)
    return out.reshape(B, S, D)


# speedup vs baseline: 2.4115x; 2.4115x over previous
import functools

import jax
import jax.numpy as jnp
from jax import lax
from jax.experimental import pallas as pl
from jax.experimental.pallas import tpu as pltpu
from jax.experimental.pallas import tpu_sc as plsc

_NC = 2
_NS = 16
_NW = _NC * _NS
_CH = 2


def _sc_body(nchunk, S, D, x_hbm, tbl_hbm, out_hbm,
             tblv, buf0, buf1, tsem, isem0, isem1, osem0, osem1):
    wid = lax.axis_index("s") * _NC + lax.axis_index("c")
    base = wid * (nchunk * _CH)
    bufs = (buf0, buf1)
    isems = (isem0, isem1)
    osems = (osem0, osem1)

    tcp = pltpu.make_async_copy(tbl_hbm, tblv, tsem)
    tcp.start()
    tcp.wait()

    def in_copy(k):
        return pltpu.make_async_copy(
            x_hbm.at[pl.ds(base + k * _CH, _CH)], bufs[k % 2], isems[k % 2]
        )

    def out_copy(k):
        return pltpu.make_async_copy(
            bufs[k % 2], out_hbm.at[pl.ds(base + k * _CH, _CH)], osems[k % 2]
        )

    in_copy(0).start()
    for k in range(nchunk):
        in_copy(k).wait()
        b = bufs[k % 2]

        @plsc.parallel_loop(0, S, 1, unroll=2)
        def _add(s):
            for jj in range(D // 16):
                t = tblv[s, pl.ds(jj * 16, 16)]
                for c in range(_CH):
                    b[c, s, pl.ds(jj * 16, 16)] = b[c, s, pl.ds(jj * 16, 16)] + t

        out_copy(k).start()
        if k + 1 < nchunk:
            if k >= 1:
                out_copy(k - 1).wait()
            in_copy(k + 1).start()
    for k in range(max(0, nchunk - 2), nchunk):
        out_copy(k).wait()


def kernel(x, pos_emb_weight):
    B, S, D = x.shape
    tbl = pos_emb_weight[:S]
    nchunk = B // (_NW * _CH)
    mesh = plsc.VectorSubcoreMesh(core_axis_name="c", subcore_axis_name="s")
    body = functools.partial(_sc_body, nchunk, S, D)
    return pl.kernel(
        body,
        out_type=jax.ShapeDtypeStruct((B, S, D), jnp.float32),
        mesh=mesh,
        scratch_types=[
            pltpu.VMEM((S, D), jnp.float32),
            pltpu.VMEM((_CH, S, D), jnp.float32),
            pltpu.VMEM((_CH, S, D), jnp.float32),
            pltpu.SemaphoreType.DMA,
            pltpu.SemaphoreType.DMA,
            pltpu.SemaphoreType.DMA,
            pltpu.SemaphoreType.DMA,
            pltpu.SemaphoreType.DMA,
        ],
    )(x, tbl)
